# baseline (speedup 1.0000x reference)
import functools

import jax
import jax.numpy as jnp
from jax import lax
from jax.experimental import pallas as pl
from jax.experimental.pallas import tpu as pltpu
from jax.experimental.pallas import tpu_sc as plsc

N = 10000
E = 320000
D = 128

NC = 2
NS = 16
NW = NC * NS
CK = 128
EC = E // CK
NCH = EC // NW
XTRA = EC - NW * NCH
NPD = 10240
RPTD = NPD // NS
NPA = 10112
RPTA = NPA // NS


def _tile_copy(src, dst, s, rpt):
    pltpu.sync_copy(src.at[pl.ds(s * rpt, rpt)], dst.at[pl.ds(s * rpt, rpt)])

@functools.cache
def _mesh():
    return plsc.VectorSubcoreMesh(core_axis_name="c", subcore_axis_name="s",
                                  num_cores=NC, num_subcores=NS)


def _sc_deg_body(ei_hbm, ew_hbm, out_hbm, col6, ew6, zb1, acc, sem_i, sem_s):
    c = lax.axis_index("c")
    s = lax.axis_index("s")
    wid = s * NC + c
    cb = wid * NCH + jnp.minimum(wid, XTRA)
    extra = wid < XTRA

    def zb_init(j, carry):
        zb1[pl.ds(j * 16, 16)] = jnp.zeros((16,), jnp.float32)
        return carry

    lax.fori_loop(0, RPTD // 16, zb_init, 0)
    pltpu.sync_copy(zb1, acc.at[pl.ds(s * RPTD, RPTD)])
    plsc.subcore_barrier()

    def fetch(i, k):
        e0 = (cb + i) * CK
        return ((ei_hbm.at[1, pl.ds(e0, CK)], col6.at[k], sem_i),
                (ew_hbm.at[pl.ds(e0, CK)], ew6.at[k], sem_i))

    def sextet(p, carry):
        for k in range(6):
            for args in fetch(p * 6 + k, k):
                pltpu.async_copy(*args)
        for k in range(6):
            for args in fetch(p * 6 + k, k):
                pltpu.make_async_copy(*args).wait()
        for k in range(6):
            pltpu.async_copy(ew6.at[k], acc.at[col6.at[k]], sem_s, add=True)
        for k in range(6):
            pltpu.make_async_copy(ew6.at[k], acc.at[col6.at[k]],
                                  sem_s).wait()
        return carry

    lax.fori_loop(0, NCH // 6, sextet, 0)

    @pl.when(extra)
    def _():
        for args in fetch(NCH, 0):
            pltpu.sync_copy(args[0], args[1])
        pltpu.sync_copy(ew6.at[0], acc.at[col6.at[0]], add=True)

    plsc.subcore_barrier()
    _tile_copy(acc, out_hbm.at[c], s, RPTD)


@functools.cache
def _sc_deg():
    return pl.kernel(
        _sc_deg_body,
        out_type=jax.ShapeDtypeStruct((NC, NPD), jnp.float32),
        mesh=_mesh(),
        scratch_types=[
            pltpu.VMEM((6, CK), jnp.int32),
            pltpu.VMEM((6, CK), jnp.float32),
            pltpu.VMEM((RPTD,), jnp.float32),
            pltpu.VMEM_SHARED((NPD,), jnp.float32),
            pltpu.SemaphoreType.DMA,
            pltpu.SemaphoreType.DMA,
        ],
    )


def _sc_agg_body(ei_hbm, ew_hbm, xws_hbm, out_hbm,
                 col_v, row_v, ew_v, rows_v, acc,
                 sem_r, sem_w, sem_c, sem_g, sem_s):
    c = lax.axis_index("c")
    s = lax.axis_index("s")
    wid = s * NC + c
    cb = wid * NCH + jnp.minimum(wid, XTRA)
    extra = wid < XTRA

    def zrow(r, carry):
        for j in range(D // 16):
            rows_v[2, r, pl.ds(j * 16, 16)] = jnp.zeros((16,), jnp.float32)
        return carry

    lax.fori_loop(0, CK, zrow, 0)
    zs = [CK] * (RPTA // CK) + ([RPTA % CK] if RPTA % CK else [])
    for t, sz in enumerate(zs):
        pltpu.async_copy(rows_v.at[2, pl.ds(0, sz)],
                         acc.at[pl.ds(s * RPTA + t * CK, sz)], sem_g.at[2])
    for t, sz in enumerate(zs):
        pltpu.make_async_copy(rows_v.at[2, pl.ds(0, sz)],
                              acc.at[pl.ds(s * RPTA + t * CK, sz)],
                              sem_g.at[2]).wait()

    def row_start(i, b):
        pltpu.async_copy(ei_hbm.at[0, pl.ds((cb + i) * CK, CK)], row_v.at[b],
                         sem_r.at[b])

    def row_wait(i, b):
        pltpu.make_async_copy(ei_hbm.at[0, pl.ds((cb + i) * CK, CK)],
                              row_v.at[b], sem_r.at[b]).wait()

    def ew_start(i, b):
        pltpu.async_copy(ew_hbm.at[pl.ds((cb + i) * CK, CK)], ew_v.at[b],
                         sem_w.at[b])

    def ew_wait(i, b):
        pltpu.make_async_copy(ew_hbm.at[pl.ds((cb + i) * CK, CK)],
                              ew_v.at[b], sem_w.at[b]).wait()

    def col_start(i, b):
        pltpu.async_copy(ei_hbm.at[1, pl.ds((cb + i) * CK, CK)], col_v.at[b],
                         sem_c.at[b])

    def col_wait(i, b):
        pltpu.make_async_copy(ei_hbm.at[1, pl.ds((cb + i) * CK, CK)],
                              col_v.at[b], sem_c.at[b]).wait()

    def gather_start(i, b3, b2):
        pltpu.async_copy(xws_hbm.at[row_v.at[b2]], rows_v.at[b3],
                         sem_g.at[b3], priority=1)

    def gather_wait(i, b3, b2):
        pltpu.make_async_copy(xws_hbm.at[row_v.at[b2]], rows_v.at[b3],
                              sem_g.at[b3]).wait()

    def scatter_start(i, b3):
        pltpu.async_copy(rows_v.at[b3], acc.at[col_v.at[b3]], sem_s.at[b3],
                         add=True)

    def scatter_wait(i, b3):
        pltpu.make_async_copy(rows_v.at[b3], acc.at[col_v.at[b3]],
                              sem_s.at[b3]).wait()

    def scale(b3, b2):
        def group(g, gcarry):
            w16 = ew_v[b2, pl.ds(g * 16, 16)]
            for l in range(16):
                e = g * 16 + l
                w = w16[l]
                for j in range(D // 16):
                    sl = pl.ds(j * 16, 16)
                    rows_v[b3, e, sl] = rows_v[b3, e, sl] * w
            return gcarry

        lax.fori_loop(0, CK // 16, group, 0)

    row_start(0, 0)
    ew_start(0, 0)
    col_start(0, 0)
    row_start(1, 1)
    ew_start(1, 1)
    plsc.subcore_barrier()
    row_wait(0, 0)
    ew_wait(0, 0)
    gather_start(0, 0, 0)

    def six(p, carry):
        for k in range(6):
            i = p * 6 + k
            b3 = k % 3
            b2 = k % 2
            nb3 = (k + 1) % 3
            nb2 = 1 - b2

            gather_wait(i, b3, b2)

            @pl.when(i + 2 < NCH)
            def _():
                row_start(i + 2, b2)

            scale(b3, b2)
            col_wait(i, b3)
            scatter_start(i, b3)

            @pl.when(i + 2 < NCH)
            def _():
                ew_start(i + 2, b2)

            @pl.when(i + 1 < NCH)
            def _():
                row_wait(i + 1, nb2)
                ew_wait(i + 1, nb2)

                @pl.when(i >= 2)
                def _():
                    scatter_wait(i - 2, nb3)

                col_start(i + 1, nb3)
                gather_start(i + 1, nb3, nb2)
        return carry

    lax.fori_loop(0, NCH // 6, six, 0)
    scatter_wait(NCH - 3, (NCH - 3) % 3)
    scatter_wait(NCH - 2, (NCH - 2) % 3)
    scatter_wait(NCH - 1, (NCH - 1) % 3)

    @pl.when(extra)
    def _():
        e0 = (cb + NCH) * CK
        pltpu.sync_copy(ei_hbm.at[0, pl.ds(e0, CK)], row_v.at[0])
        pltpu.sync_copy(ei_hbm.at[1, pl.ds(e0, CK)], col_v.at[0])
        pltpu.sync_copy(ew_hbm.at[pl.ds(e0, CK)], ew_v.at[0])
        pltpu.async_copy(xws_hbm.at[row_v.at[0]], rows_v.at[0],
                         sem_g.at[0]).wait()
        scale(0, 0)
        pltpu.sync_copy(rows_v.at[0], acc.at[col_v.at[0]], add=True)

    plsc.subcore_barrier()
    _tile_copy(acc, out_hbm.at[c], s, RPTA)


@functools.cache
def _sc_agg():
    return pl.kernel(
        _sc_agg_body,
        out_type=jax.ShapeDtypeStruct((NC, NPA, D), jnp.float32),
        mesh=_mesh(),
        scratch_types=[
            pltpu.VMEM((3, CK), jnp.int32),
            pltpu.VMEM((2, CK), jnp.int32),
            pltpu.VMEM((2, CK), jnp.float32),
            pltpu.VMEM((3, CK, D), jnp.float32),
            pltpu.VMEM_SHARED((NPA, D), jnp.float32),
            pltpu.SemaphoreType.DMA((2,)),
            pltpu.SemaphoreType.DMA((2,)),
            pltpu.SemaphoreType.DMA((3,)),
            pltpu.SemaphoreType.DMA((3,)),
            pltpu.SemaphoreType.DMA((3,)),
        ],
    )


def _tc_a_body(degp_ref, x_ref, w1_ref, xws_ref, dis_ref):
    deg = degp_ref[:N, 0:1] + degp_ref[:N, 1:2] + 1.0
    dis = lax.rsqrt(deg)
    xw = lax.dot_general(x_ref[...], w1_ref[...],
                         (((1,), (1,)), ((), ())),
                         preferred_element_type=jnp.float32)
    xws_ref[...] = xw * dis
    dis_ref[...] = dis


def _tc_a(degp, x, W1):
    return pl.pallas_call(
        _tc_a_body,
        out_shape=(jax.ShapeDtypeStruct((N, D), jnp.float32),
                   jax.ShapeDtypeStruct((N, 1), jnp.float32)),
    )(degp, x, W1)


def _bn_cols(a, gamma, beta):
    mu = jnp.mean(a, axis=0, keepdims=True)
    d = a - mu
    var = jnp.mean(d * d, axis=0, keepdims=True)
    return d * lax.rsqrt(var + 1e-5) * gamma + beta


def _tc_b_body(aggp_ref, xws_ref, dis_ref, b1_ref, g1_ref, be1_ref, w2_ref,
               h_ref, xws2_ref):
    dis = dis_ref[...]
    a = dis * (aggp_ref[0, :N] + aggp_ref[1, :N] + xws_ref[...]) + b1_ref[...]
    a = jnp.maximum(a, 0.0)
    h = _bn_cols(a, g1_ref[...], be1_ref[...])
    h_ref[...] = h
    xw2 = lax.dot_general(h, w2_ref[...], (((1,), (1,)), ((), ())),
                          preferred_element_type=jnp.float32)
    xws2_ref[...] = xw2 * dis


def _tc_b(aggp, xws1, dis, b1, gamma1, beta1, W2):
    return pl.pallas_call(
        _tc_b_body,
        out_shape=(jax.ShapeDtypeStruct((N, D), jnp.float32),
                   jax.ShapeDtypeStruct((N, D), jnp.float32)),
    )(aggp, xws1, dis, b1.reshape(1, D), gamma1.reshape(1, D),
      beta1.reshape(1, D), W2)


def _tc_c_body(aggp_ref, xws_ref, dis_ref, b2_ref, g2_ref, be2_ref,
               h_ref, x_ref, wih1_ref, bsum1_ref, wih2_ref, bsum2_ref,
               wlin_ref, blin_ref, out_ref):
    dis = dis_ref[...]
    a = dis * (aggp_ref[0, :N] + aggp_ref[1, :N] + xws_ref[...]) + b2_ref[...]
    a = jnp.maximum(a, 0.0)
    h2 = _bn_cols(a, g2_ref[...], be2_ref[...])
    h = h_ref[...]

    wih1 = jnp.concatenate([wih1_ref[:D], wih1_ref[2 * D:]], axis=0)
    bs1 = jnp.concatenate([bsum1_ref[:, :D], bsum1_ref[:, 2 * D:]], axis=1)
    g1 = (lax.dot_general(h, wih1[:, :D], (((1,), (1,)), ((), ())),
                          preferred_element_type=jnp.float32)
          + lax.dot_general(h2, wih1[:, D:], (((1,), (1,)), ((), ())),
                            preferred_element_type=jnp.float32)
          + bs1)
    i1 = jax.nn.sigmoid(g1[:, :D])
    gg1 = jnp.tanh(g1[:, D:2 * D])
    o1 = jax.nn.sigmoid(g1[:, 2 * D:])
    H1 = o1 * jnp.tanh(i1 * gg1)

    wih2 = jnp.concatenate([wih2_ref[:D], wih2_ref[2 * D:]], axis=0)
    bs2 = jnp.concatenate([bsum2_ref[:, :D], bsum2_ref[:, 2 * D:]], axis=1)
    g2 = (lax.dot_general(H1, wih2, (((1,), (1,)), ((), ())),
                          preferred_element_type=jnp.float32)
          + bs2)
    i2 = jax.nn.sigmoid(g2[:, :D])
    gg2 = jnp.tanh(g2[:, D:2 * D])
    o2 = jax.nn.sigmoid(g2[:, 2 * D:])
    H2 = o2 * jnp.tanh(i2 * gg2)

    wlin = wlin_ref[...]
    Hr1 = jnp.maximum(H1, 0.0)
    Hr2 = jnp.maximum(H2, 0.0)
    Hr3 = jnp.maximum(x_ref[...], 0.0)
    o = (lax.dot_general(Hr1, wlin[:, :D], (((1,), (1,)), ((), ())),
                         preferred_element_type=jnp.float32)
         + lax.dot_general(Hr2, wlin[:, D:2 * D], (((1,), (1,)), ((), ())),
                           preferred_element_type=jnp.float32)
         + lax.dot_general(Hr3, wlin[:, 2 * D:], (((1,), (1,)), ((), ())),
                           preferred_element_type=jnp.float32)
         + blin_ref[...])
    out_ref[...] = jnp.tanh(o)


def _tc_c(aggp, xws2, dis, b2, gamma2, beta2, h, x,
          Wih1, bsum1, Wih2, bsum2, Wlin, blin):
    return pl.pallas_call(
        _tc_c_body,
        out_shape=jax.ShapeDtypeStruct((N, 1), jnp.float32),
    )(aggp, xws2, dis, b2.reshape(1, D), gamma2.reshape(1, D),
      beta2.reshape(1, D), h, x, Wih1, bsum1.reshape(1, 4 * D),
      Wih2, bsum2.reshape(1, 4 * D), Wlin, blin.reshape(1, 1))


def kernel(x, edge_index, edge_weight, W1, b1, gamma1, beta1,
           W2, b2, gamma2, beta2, Wih1, Whh1, bih1, bhh1,
           Wih2, Whh2, bih2, bhh2, Wlin, blin):
    degp = _sc_deg()(edge_index, edge_weight)
    xws1, dis = _tc_a(degp.T, x, W1)
    aggp1 = _sc_agg()(edge_index, edge_weight, xws1)
    h, xws2 = _tc_b(aggp1, xws1, dis, b1, gamma1, beta1, W2)
    aggp2 = _sc_agg()(edge_index, edge_weight, xws2)
    out = _tc_c(aggp2, xws2, dis, b2, gamma2, beta2, h, x,
                Wih1, bih1 + bhh1, Wih2, bih2 + bhh2, Wlin, blin)
    return out
